# baseline (speedup 1.0000x reference)
import functools

import jax
import jax.numpy as jnp
from jax.experimental import pallas as pl
from jax.experimental.pallas import tpu as pltpu

NUM_CB = 4
K = 1024
D = 256
N = 16384
BT = 1024
GRID = N // BT

_DN = (((1,), (0,)), ((), ()))


def _stage_body(r, rn, cn, hi, cb):
    dots = jax.lax.dot_general(
        r.astype(jnp.bfloat16), hi, (((1,), (1,)), ((), ())),
        preferred_element_type=jnp.float32)
    dists = (rn + cn) - 2.0 * dots
    idx = jnp.argmin(dists, axis=1).astype(jnp.int32)
    onehot = (jax.lax.broadcasted_iota(jnp.int32, (BT, K), 1)
              == idx[:, None]).astype(jnp.float32)
    zqi = jax.lax.dot_general(onehot, cb, _DN,
                              preferred_element_type=jnp.float32,
                              precision=jax.lax.Precision.HIGHEST)
    r_new = r - zqi
    return idx, r_new


def _loss_write(loss_ref, r_new):
    lane = jax.lax.broadcasted_iota(jnp.int32, (1, 1, 8), 2)
    loss_ref[...] = jnp.where(lane == 0, jnp.sum(r_new * r_new), 0.0)


def _stage_mid_kernel(r_ref, rn_ref, cn_ref, hi_ref, cb_ref,
                      idx_ref, rnew_ref, loss_ref):
    idx, r_new = _stage_body(r_ref[...], rn_ref[...], cn_ref[...],
                             hi_ref[...], cb_ref[...])
    idx_ref[...] = idx[:, None]
    rnew_ref[...] = r_new
    _loss_write(loss_ref, r_new)


def _stage_last_kernel(z_ref, r_ref, rn_ref, cn_ref, hi_ref, cb_ref,
                       idx_ref, zq_ref, loss_ref):
    idx, r_new = _stage_body(r_ref[...], rn_ref[...], cn_ref[...],
                             hi_ref[...], cb_ref[...])
    idx_ref[...] = idx[:, None]
    z = z_ref[...]
    zq_ref[...] = z - r_new
    _loss_write(loss_ref, r_new)


def _row_spec():
    return pl.BlockSpec((BT, D), lambda i: (i, 0))


def _full_spec(shape):
    nd = len(shape)
    return pl.BlockSpec(shape, lambda i: (0,) * nd)


def _stage_call(r, rn, cn, hi, cb, *, z=None):
    last = z is not None
    kern = _stage_last_kernel if last else _stage_mid_kernel
    in_specs = [_row_spec(),
                pl.BlockSpec((BT, 1), lambda i: (i, 0)),
                _full_spec((1, K)),
                _full_spec((K, D)),
                _full_spec((K, D))]
    if last:
        in_specs = [_row_spec()] + in_specs
    out_specs = [pl.BlockSpec((BT, 1), lambda i: (i, 0)),
                 _row_spec(),
                 pl.BlockSpec((1, 1, 8), lambda i: (i, 0, 0))]
    out_shape = [jax.ShapeDtypeStruct((N, 1), jnp.int32),
                 jax.ShapeDtypeStruct((N, D), jnp.float32),
                 jax.ShapeDtypeStruct((GRID, 1, 8), jnp.float32)]
    args = (z, r, rn, cn, hi, cb) if last else (r, rn, cn, hi, cb)
    return pl.pallas_call(
        kern,
        grid=(GRID,),
        in_specs=in_specs,
        out_specs=out_specs,
        out_shape=out_shape,
    )(*args)


@jax.jit
def kernel(z, codebooks):
    hi = codebooks.astype(jnp.bfloat16)
    cns = [jnp.sum(codebooks[s] ** 2, axis=1)[None, :] for s in range(NUM_CB)]

    r = z
    codes, loss_parts = [], []
    for s in range(NUM_CB):
        rn = jnp.sum(r ** 2, axis=1, keepdims=True)
        if s < NUM_CB - 1:
            idx, r, lp = _stage_call(r, rn, cns[s], hi[s], codebooks[s])
        else:
            idx, zq, lp = _stage_call(r, rn, cns[s], hi[s], codebooks[s], z=z)
        codes.append(idx)
        loss_parts.append(lp)
    codes = jnp.concatenate(codes, axis=1)
    loss = sum(jnp.sum(lp) for lp in loss_parts) / jnp.float32(N * D)
    zq_st = z + (zq - z)
    return zq_st, codes, loss, loss


# speedup vs baseline: 1.1875x; 1.1875x over previous
import functools

import jax
import jax.numpy as jnp
from jax import lax
from jax.experimental import pallas as pl
from jax.experimental.pallas import tpu as pltpu
from jax.experimental.pallas import tpu_sc as plsc

NUM_CB = 4
K = 1024
D = 256
N = 16384
BT = 1024
GRID = N // BT

NW = 32
B_PER_W = N // NW
CHUNK = 256
NCHUNK = B_PER_W // CHUNK

_sc_mesh = plsc.VectorSubcoreMesh(core_axis_name="c", subcore_axis_name="s")


@functools.partial(
    pl.kernel, mesh=_sc_mesh,
    out_type=jax.ShapeDtypeStruct((N, D), jnp.float32),
    scratch_types=[
        pltpu.VMEM((CHUNK,), jnp.int32),
        pltpu.VMEM((CHUNK, D), jnp.float32),
        pltpu.SemaphoreType.DMA,
    ],
)
def _sc_gather(table_hbm, idx_hbm, out_hbm, idx_v, rows_v, sem):
    wid = lax.axis_index("s") * 2 + lax.axis_index("c")
    base = wid * B_PER_W
    for c in range(NCHUNK):
        off = base + c * CHUNK
        pltpu.sync_copy(idx_hbm.at[pl.ds(off, CHUNK)], idx_v)
        pltpu.async_copy(table_hbm.at[idx_v], rows_v, sem).wait()
        pltpu.sync_copy(rows_v, out_hbm.at[pl.ds(off, CHUNK)])


def _argmin_idx(r, rn, cn, hi):
    dots = jax.lax.dot_general(
        r.astype(jnp.bfloat16), hi, (((1,), (1,)), ((), ())),
        preferred_element_type=jnp.float32)
    dists = (rn + cn) - 2.0 * dots
    return jnp.argmin(dists, axis=1).astype(jnp.int32)


def _loss_write(loss_ref, r_new):
    lane = jax.lax.broadcasted_iota(jnp.int32, (1, 1, 8), 2)
    loss_ref[...] = jnp.where(lane == 0, jnp.sum(r_new * r_new), 0.0)


def _first_kernel(z_ref, rn_ref, cn_ref, hi_ref, idx_ref):
    idx = _argmin_idx(z_ref[...], rn_ref[...], cn_ref[...], hi_ref[...])
    idx_ref[...] = idx[:, None]


def _mid_kernel(rprev_ref, zqi_ref, rn_ref, cn_ref, hi_ref,
                idx_ref, r_ref, loss_ref):
    r = rprev_ref[...] - zqi_ref[...]
    idx = _argmin_idx(r, rn_ref[...], cn_ref[...], hi_ref[...])
    idx_ref[...] = idx[:, None]
    r_ref[...] = r
    _loss_write(loss_ref, r)


def _final_kernel(z_ref, rprev_ref, zqi_ref, zq_ref, loss_ref):
    r = rprev_ref[...] - zqi_ref[...]
    zq_ref[...] = z_ref[...] - r
    _loss_write(loss_ref, r)


def _row_spec():
    return pl.BlockSpec((BT, D), lambda i: (i, 0))


def _rn_spec():
    return pl.BlockSpec((BT, 1), lambda i: (i, 0))


def _full_spec(shape):
    nd = len(shape)
    return pl.BlockSpec(shape, lambda i: (0,) * nd)


_IDX_OUT = (pl.BlockSpec((BT, 1), lambda i: (i, 0)),
            jax.ShapeDtypeStruct((N, 1), jnp.int32))
_LOSS_OUT = (pl.BlockSpec((1, 1, 8), lambda i: (i, 0, 0)),
             jax.ShapeDtypeStruct((GRID, 1, 8), jnp.float32))
_ROW_OUT = (pl.BlockSpec((BT, D), lambda i: (i, 0)),
            jax.ShapeDtypeStruct((N, D), jnp.float32))


def _first_call(z, rn, cn, hi):
    return pl.pallas_call(
        _first_kernel, grid=(GRID,),
        in_specs=[_row_spec(), _rn_spec(), _full_spec((1, K)),
                  _full_spec((K, D))],
        out_specs=[_IDX_OUT[0]], out_shape=[_IDX_OUT[1]],
    )(z, rn, cn, hi)[0]


def _mid_call(rprev, zqi, rn, cn, hi):
    return pl.pallas_call(
        _mid_kernel, grid=(GRID,),
        in_specs=[_row_spec(), _row_spec(), _rn_spec(), _full_spec((1, K)),
                  _full_spec((K, D))],
        out_specs=[_IDX_OUT[0], _ROW_OUT[0], _LOSS_OUT[0]],
        out_shape=[_IDX_OUT[1], _ROW_OUT[1], _LOSS_OUT[1]],
    )(rprev, zqi, rn, cn, hi)


def _final_call(z, rprev, zqi):
    return pl.pallas_call(
        _final_kernel, grid=(GRID,),
        in_specs=[_row_spec(), _row_spec(), _row_spec()],
        out_specs=[_ROW_OUT[0], _LOSS_OUT[0]],
        out_shape=[_ROW_OUT[1], _LOSS_OUT[1]],
    )(z, rprev, zqi)


@jax.jit
def kernel(z, codebooks):
    hi = codebooks.astype(jnp.bfloat16)
    cns = [jnp.sum(codebooks[s] ** 2, axis=1)[None, :] for s in range(NUM_CB)]

    rn = jnp.sum(z ** 2, axis=1, keepdims=True)
    idx0 = _first_call(z, rn, cns[0], hi[0])
    zqi = _sc_gather(codebooks[0], idx0.reshape(N))

    codes, loss_parts = [idx0], []
    rprev = z
    for s in range(1, NUM_CB):
        rn = jnp.sum((rprev - zqi) ** 2, axis=1, keepdims=True)
        idx, r, lp = _mid_call(rprev, zqi, rn, cns[s], hi[s])
        codes.append(idx)
        loss_parts.append(lp)
        rprev, zqi = r, _sc_gather(codebooks[s], idx.reshape(N))
    zq, lp = _final_call(z, rprev, zqi)
    loss_parts.append(lp)

    codes = jnp.concatenate(codes, axis=1)
    loss = sum(jnp.sum(p) for p in loss_parts) / jnp.float32(N * D)
    zq_st = z + (zq - z)
    return zq_st, codes, loss, loss
